# Initial kernel scaffold; baseline (speedup 1.0000x reference)
import jax
import jax.numpy as jnp
from jax.experimental import pallas as pl


def kernel(x, edge_index, edge_attr, batch, x_lin_W, x_lin_b, edge_table, W1, b1, W2, b2, bn_g, bn_b, lin1_W, lin1_b, lin2_W, lin2_b):
    raise NotImplementedError("write your pallas kernel here")


# speedup vs baseline: 2.6182x; 2.6182x over previous
import functools

import jax
import jax.numpy as jnp
from jax import lax
from jax.experimental import pallas as pl
from jax.experimental.pallas import tpu as pltpu
from jax.experimental.pallas import tpu_sc as plsc

N = 10000
E = 320000
D = 128
G = 64
OUT = 10
NA = 4

NC = 2
NS = 16
NWK = NC * NS
CH = 128
EPW = 10240
E_PAD = EPW * NWK
NCHUNK = EPW // CH
SP_ROWS = 10240
ZP = SP_ROWS // NS
CP = 624
CP_TAIL = N - (NS - 1) * CP - CP

_f32 = jnp.float32


def _sc_edge_body(rtab_hbm, src_hbm, attr_hbm, dst_hbm, out_hbm,
                  gsrc_v, gattr_v, gidx_v, gdst_v, rows_v, zrow_v, agg_sh, sem):
    cid = lax.axis_index("c")
    sid = lax.axis_index("s")
    wid = sid * NC + cid

    zv = jnp.zeros((16,), _f32)
    for r in range(16):
        for j in range(D // 16):
            zrow_v[r, pl.ds(16 * j, 16)] = zv

    def zbody(i, c):
        pltpu.sync_copy(zrow_v, agg_sh.at[pl.ds(sid * ZP + i * 16, 16)])
        return c

    lax.fori_loop(0, ZP // 16, zbody, 0)
    plsc.subcore_barrier()

    base0 = wid * EPW

    def ebody(ci, c):
        b = base0 + ci * CH
        pltpu.sync_copy(src_hbm.at[pl.ds(b, CH)], gsrc_v)
        pltpu.sync_copy(attr_hbm.at[pl.ds(b, CH)], gattr_v)
        pltpu.sync_copy(dst_hbm.at[pl.ds(b, CH)], gdst_v)
        for j in range(CH // 16):
            s16 = gsrc_v[pl.ds(16 * j, 16)]
            a16 = gattr_v[pl.ds(16 * j, 16)]
            gidx_v[pl.ds(16 * j, 16)] = a16 * N + s16
        pltpu.async_copy(rtab_hbm.at[gidx_v], rows_v, sem).wait()
        pltpu.sync_copy(rows_v, agg_sh.at[gdst_v], add=True)
        return c

    lax.fori_loop(0, NCHUNK, ebody, 0)
    plsc.subcore_barrier()

    pltpu.sync_copy(agg_sh.at[pl.ds(sid * CP, CP)],
                    out_hbm.at[pl.ds(cid * N + sid * CP, CP)])

    @pl.when(sid == NS - 1)
    def _tail():
        pltpu.sync_copy(agg_sh.at[pl.ds(NS * CP, CP_TAIL)],
                        out_hbm.at[pl.ds(cid * N + NS * CP, CP_TAIL)])


@functools.cache
def _get_sc_edge():
  return pl.kernel(
    _sc_edge_body,
    out_type=jax.ShapeDtypeStruct((NC * N, D), _f32),
    mesh=plsc.VectorSubcoreMesh(core_axis_name="c", subcore_axis_name="s",
                                num_cores=NC, num_subcores=NS),
    scratch_types=[
        pltpu.VMEM((CH,), jnp.int32),
        pltpu.VMEM((CH,), jnp.int32),
        pltpu.VMEM((CH,), jnp.int32),
        pltpu.VMEM((CH,), jnp.int32),
        pltpu.VMEM((CH, D), _f32),
        pltpu.VMEM((16, D), _f32),
        pltpu.VMEM_SHARED((SP_ROWS, D), _f32),
        pltpu.SemaphoreType.DMA,
    ],
  )


def _enc_body(x_ref, w_ref, b_ref, t_ref, h_ref, rtab_ref):
    h = jnp.dot(x_ref[...], w_ref[...], preferred_element_type=_f32) + b_ref[...]
    h_ref[...] = h
    for a in range(NA):
        rtab_ref[pl.ds(a * N, N), :] = jnp.maximum(h + t_ref[pl.ds(a, 1), :], 0.0)


_enc = pl.pallas_call(
    _enc_body,
    out_shape=[jax.ShapeDtypeStruct((N, D), _f32),
               jax.ShapeDtypeStruct((NA * N, D), _f32)],
)


def _dense_body(h_ref, agg_ref, w1_ref, b1_ref, w2_ref, b2_ref, g_ref, bb_ref,
                t_ref, ho_ref, rtab_ref, *, last):
    z = h_ref[...] + agg_ref[pl.ds(0, N), :] + agg_ref[pl.ds(N, N), :]
    z = jnp.maximum(jnp.dot(z, w1_ref[...], preferred_element_type=_f32)
                    + b1_ref[...], 0.0)
    z = jnp.maximum(jnp.dot(z, w2_ref[...], preferred_element_type=_f32)
                    + b2_ref[...], 0.0)
    m = jnp.mean(z, axis=0, keepdims=True)
    zc = z - m
    v = jnp.mean(zc * zc, axis=0, keepdims=True)
    hn = zc * lax.rsqrt(v + 1e-5) * g_ref[...] + bb_ref[...]
    ho_ref[...] = hn
    if not last:
        for a in range(NA):
            rtab_ref[pl.ds(a * N, N), :] = jnp.maximum(
                hn + t_ref[pl.ds(a, 1), :], 0.0)


_dense_mid = pl.pallas_call(
    functools.partial(_dense_body, last=False),
    out_shape=[jax.ShapeDtypeStruct((N, D), _f32),
               jax.ShapeDtypeStruct((NA * N, D), _f32)],
)


def _dense_last_body(h_ref, agg_ref, w1_ref, b1_ref, w2_ref, b2_ref, g_ref,
                     bb_ref, t_ref, ho_ref):
    _dense_body(h_ref, agg_ref, w1_ref, b1_ref, w2_ref, b2_ref, g_ref, bb_ref,
                t_ref, ho_ref, None, last=True)


_dense_last = pl.pallas_call(
    _dense_last_body,
    out_shape=[jax.ShapeDtypeStruct((N, D), _f32)],
)


def _readout_body(h_ref, brow_ref, bcol_ref, w1_ref, b1_ref, w2_ref, b2_ref,
                  o_ref):
    h = h_ref[...]
    brow = brow_ref[...]
    gids = lax.broadcasted_iota(jnp.int32, (G, 1), 0)
    onehot = (gids == brow).astype(_f32)
    dn = (((1,), (0,)), ((), ()))
    sums = lax.dot_general(onehot, h, dn, preferred_element_type=_f32)
    cntb = lax.dot_general(onehot, jnp.ones_like(h), dn,
                           preferred_element_type=_f32)
    meanp = sums / jnp.maximum(cntb, 1.0)
    bcol = bcol_ref[...]
    neg = jnp.float32(-3.0e38)
    rows = []
    for gg in range(G):
        mg = jnp.where(bcol == gg, h, neg)
        rows.append(jnp.max(mg, axis=0, keepdims=True))
    maxp = jnp.concatenate(rows, axis=0)
    maxp = jnp.where(cntb > 0.0, maxp, 0.0)
    gemb = jnp.concatenate([meanp, maxp], axis=1)
    hid = jnp.maximum(jnp.dot(gemb, w1_ref[...], preferred_element_type=_f32)
                      + b1_ref[...], 0.0)
    o_ref[...] = jnp.dot(hid, w2_ref[...], preferred_element_type=_f32) + b2_ref[...]


_readout = pl.pallas_call(
    _readout_body,
    out_shape=jax.ShapeDtypeStruct((G, OUT), _f32),
)


def kernel(x, edge_index, edge_attr, batch, x_lin_W, x_lin_b, edge_table,
           W1, b1, W2, b2, bn_g, bn_b, lin1_W, lin1_b, lin2_W, lin2_b):
    src = edge_index[0].astype(jnp.int32)
    dst = edge_index[1].astype(jnp.int32)
    attr = edge_attr.astype(jnp.int32)
    pad = E_PAD - E
    zpad = jnp.zeros((pad,), jnp.int32)
    srcp = jnp.concatenate([src, zpad])
    attrp = jnp.concatenate([attr, zpad])
    dstp = jnp.concatenate([dst, jnp.full((pad,), N, jnp.int32)])

    xb = x_lin_b.reshape(1, D)
    h, rtab = _enc(x, x_lin_W, xb, edge_table)
    for l in range(3):
        agg = _get_sc_edge()(rtab, srcp, attrp, dstp)
        args = (h, agg, W1[l], b1[l].reshape(1, D), W2[l], b2[l].reshape(1, D),
                bn_g[l].reshape(1, D), bn_b[l].reshape(1, D), edge_table)
        if l < 2:
            h, rtab = _dense_mid(*args)
        else:
            (h,) = _dense_last(*args)

    brow = batch.astype(jnp.int32).reshape(1, N)
    bcol = batch.astype(jnp.int32).reshape(N, 1)
    return _readout(h, brow, bcol, lin1_W, lin1_b.reshape(1, D), lin2_W,
                    lin2_b.reshape(1, OUT))
